# baseline (speedup 1.0000x reference)
import functools

import jax
import jax.numpy as jnp
from jax import lax
from jax.experimental import pallas as pl
from jax.experimental.pallas import tpu as pltpu
from jax.experimental.pallas import tpu_sc as plsc

_NC = 2
_NS = 16
_NW = _NC * _NS

_B = 16384
_D = 64
_BPW = _B // _NW


def _sc_gather(user_ids, item_ids, user_table, item_table):
  mesh = plsc.VectorSubcoreMesh(
      core_axis_name="c", subcore_axis_name="s",
      num_cores=_NC, num_subcores=_NS)

  @functools.partial(
      pl.kernel,
      out_type=[
          jax.ShapeDtypeStruct((_B, _D), jnp.float32),
          jax.ShapeDtypeStruct((_B, _D), jnp.float32),
      ],
      mesh=mesh,
      compiler_params=pltpu.CompilerParams(use_tc_tiling_on_sc=False),
      scratch_types=[
          pltpu.VMEM((_BPW,), jnp.int32),
          pltpu.VMEM((_BPW, _D), jnp.float32),
          pltpu.VMEM((_BPW,), jnp.int32),
          pltpu.VMEM((_BPW, _D), jnp.float32),
          pltpu.SemaphoreType.DMA,
          pltpu.SemaphoreType.DMA,
      ],
  )
  def gather_kernel(uid_hbm, iid_hbm, ut_hbm, it_hbm, ue_out, ie_out,
                    uidx_v, urows_v, iidx_v, irows_v, usem, isem):
    wid = lax.axis_index("s") * _NC + lax.axis_index("c")
    base = wid * _BPW
    pltpu.sync_copy(uid_hbm.at[pl.ds(base, _BPW)], uidx_v)
    pltpu.sync_copy(iid_hbm.at[pl.ds(base, _BPW)], iidx_v)
    cu = pltpu.async_copy(ut_hbm.at[uidx_v], urows_v, usem)
    ci = pltpu.async_copy(it_hbm.at[iidx_v], irows_v, isem)
    cu.wait()
    pltpu.sync_copy(urows_v, ue_out.at[pl.ds(base, _BPW)])
    ci.wait()
    pltpu.sync_copy(irows_v, ie_out.at[pl.ds(base, _BPW)])

  return gather_kernel(user_ids, item_ids, user_table, item_table)


def _mlp_body(ue_ref, ie_ref, w0a_ref, w0b_ref, b0_ref, w1_ref, b1_ref,
              w2_ref, b2_ref, wout_ref, bout_ref, out_ref):
  h = jnp.dot(ue_ref[...], w0a_ref[...], preferred_element_type=jnp.float32)
  h += jnp.dot(ie_ref[...], w0b_ref[...], preferred_element_type=jnp.float32)
  h = jnp.maximum(h + b0_ref[...], 0.0)
  h = jnp.dot(h, w1_ref[...], preferred_element_type=jnp.float32)
  h = jnp.maximum(h + b1_ref[...], 0.0)
  h = jnp.dot(h, w2_ref[...], preferred_element_type=jnp.float32)
  h = jnp.maximum(h + b2_ref[...], 0.0)
  p = jnp.dot(h, wout_ref[...], preferred_element_type=jnp.float32)
  out_ref[...] = jax.nn.sigmoid(p + bout_ref[...])


_MLP_BLK = 2048


def _tc_mlp(ue, ie, w0a, w0b, b0, w1, b1, w2, b2, wout, bout):
  grid = _B // _MLP_BLK
  full = lambda shape: pl.BlockSpec(shape, lambda i: (0,) * len(shape))
  return pl.pallas_call(
      _mlp_body,
      grid=(grid,),
      in_specs=[
          pl.BlockSpec((_MLP_BLK, _D), lambda i: (i, 0)),
          pl.BlockSpec((_MLP_BLK, _D), lambda i: (i, 0)),
          full(w0a.shape), full(w0b.shape), full(b0.shape),
          full(w1.shape), full(b1.shape),
          full(w2.shape), full(b2.shape),
          full(wout.shape), full(bout.shape),
      ],
      out_specs=pl.BlockSpec((_MLP_BLK, 1), lambda i: (i, 0)),
      out_shape=jax.ShapeDtypeStruct((_B, 1), jnp.float32),
  )(ue, ie, w0a, w0b, b0, w1, b1, w2, b2, wout, bout)


def kernel(user_ids, item_ids, user_table, item_table,
           W0, b0, W1, b1, W2, b2, Wout, bout):
  user_ids = user_ids.astype(jnp.int32)
  item_ids = item_ids.astype(jnp.int32)
  ue, ie = _sc_gather(user_ids, item_ids, user_table, item_table)
  w0a = W0[:_D]
  w0b = W0[_D:]
  return _tc_mlp(ue, ie, w0a, w0b,
                 b0.reshape(1, -1), W1, b1.reshape(1, -1),
                 W2, b2.reshape(1, -1), Wout, bout.reshape(1, 1))


# speedup vs baseline: 1.5741x; 1.5741x over previous
import functools

import jax
import jax.numpy as jnp
from jax import lax
from jax.experimental import pallas as pl
from jax.experimental.pallas import tpu as pltpu
from jax.experimental.pallas import tpu_sc as plsc

_NC = 2
_NS = 16
_NW = _NC * _NS

_B = 16384
_D = 64
_BPW = _B // _NW
_CHUNK = 256


def _sc_gather(user_ids, item_ids, user_table, item_table):
  mesh = plsc.VectorSubcoreMesh(
      core_axis_name="c", subcore_axis_name="s",
      num_cores=_NC, num_subcores=_NS)

  @functools.partial(
      pl.kernel,
      out_type=[
          jax.ShapeDtypeStruct((_B, _D), jnp.float32),
          jax.ShapeDtypeStruct((_B, _D), jnp.float32),
      ],
      mesh=mesh,
      scratch_types=[
          pltpu.VMEM((_CHUNK, _D), jnp.float32),
          pltpu.VMEM((_CHUNK, _D), jnp.float32),
          pltpu.VMEM((_BPW,), jnp.int32),
          pltpu.VMEM((_BPW,), jnp.int32),
          pltpu.SemaphoreType.DMA,
          pltpu.SemaphoreType.DMA,
      ],
  )
  def gather_kernel(uid_hbm, iid_hbm, ut_hbm, it_hbm, ue_out, ie_out,
                    urows_v, irows_v, uidx_s, iidx_s, usem, isem):
    wid = lax.axis_index("s") * _NC + lax.axis_index("c")
    base = wid * _BPW
    pltpu.sync_copy(uid_hbm.at[pl.ds(base, _BPW)], uidx_s)
    pltpu.sync_copy(iid_hbm.at[pl.ds(base, _BPW)], iidx_s)

    for c in range(_BPW // _CHUNK):
      off = c * _CHUNK

      def row_dma(g, _):
        uvec = uidx_s[pl.ds(off + g * 16, 16)]
        ivec = iidx_s[pl.ds(off + g * 16, 16)]
        for j in range(16):
          pltpu.async_copy(ut_hbm.at[uvec[j]], urows_v.at[g * 16 + j], usem)
          pltpu.async_copy(it_hbm.at[ivec[j]], irows_v.at[g * 16 + j], isem)
        return 0

      lax.fori_loop(0, _CHUNK // 16, row_dma, 0)
      pltpu.make_async_copy(ut_hbm.at[pl.ds(0, _CHUNK)], urows_v, usem).wait()
      pltpu.sync_copy(urows_v, ue_out.at[pl.ds(base + off, _CHUNK)])
      pltpu.make_async_copy(it_hbm.at[pl.ds(0, _CHUNK)], irows_v, isem).wait()
      pltpu.sync_copy(irows_v, ie_out.at[pl.ds(base + off, _CHUNK)])

  return gather_kernel(user_ids, item_ids, user_table, item_table)


def _mlp_body(ue_ref, ie_ref, w0a_ref, w0b_ref, b0_ref, w1_ref, b1_ref,
              w2_ref, b2_ref, wout_ref, bout_ref, out_ref):
  h = jnp.dot(ue_ref[...], w0a_ref[...], preferred_element_type=jnp.float32)
  h += jnp.dot(ie_ref[...], w0b_ref[...], preferred_element_type=jnp.float32)
  h = jnp.maximum(h + b0_ref[...], 0.0)
  h = jnp.dot(h, w1_ref[...], preferred_element_type=jnp.float32)
  h = jnp.maximum(h + b1_ref[...], 0.0)
  h = jnp.dot(h, w2_ref[...], preferred_element_type=jnp.float32)
  h = jnp.maximum(h + b2_ref[...], 0.0)
  p = jnp.dot(h, wout_ref[...], preferred_element_type=jnp.float32)
  out_ref[...] = jax.nn.sigmoid(p + bout_ref[...])


_MLP_BLK = 2048


def _tc_mlp(ue, ie, w0a, w0b, b0, w1, b1, w2, b2, wout, bout):
  grid = _B // _MLP_BLK
  full = lambda shape: pl.BlockSpec(shape, lambda i: (0,) * len(shape))
  return pl.pallas_call(
      _mlp_body,
      grid=(grid,),
      in_specs=[
          pl.BlockSpec((_MLP_BLK, _D), lambda i: (i, 0)),
          pl.BlockSpec((_MLP_BLK, _D), lambda i: (i, 0)),
          full(w0a.shape), full(w0b.shape), full(b0.shape),
          full(w1.shape), full(b1.shape),
          full(w2.shape), full(b2.shape),
          full(wout.shape), full(bout.shape),
      ],
      out_specs=pl.BlockSpec((_MLP_BLK, 1), lambda i: (i, 0)),
      out_shape=jax.ShapeDtypeStruct((_B, 1), jnp.float32),
  )(ue, ie, w0a, w0b, b0, w1, b1, w2, b2, wout, bout)


def kernel(user_ids, item_ids, user_table, item_table,
           W0, b0, W1, b1, W2, b2, Wout, bout):
  user_ids = user_ids.astype(jnp.int32)
  item_ids = item_ids.astype(jnp.int32)
  ue, ie = _sc_gather(user_ids, item_ids, user_table, item_table)
  w0a = W0[:_D]
  w0b = W0[_D:]
  return _tc_mlp(ue, ie, w0a, w0b,
                 b0.reshape(1, -1), W1, b1.reshape(1, -1),
                 W2, b2.reshape(1, -1), Wout, bout.reshape(1, 1))
